# Initial kernel scaffold; baseline (speedup 1.0000x reference)
import jax
import jax.numpy as jnp
from jax.experimental import pallas as pl


def kernel(user, item, hist, dense, E_user, E_item, E_hist, W_dense, b_dense, W1, b1, W2, b2):
    raise NotImplementedError("write your pallas kernel here")


# speedup vs baseline: 1.2540x; 1.2540x over previous
import functools

import jax
import jax.numpy as jnp
from jax import lax
from jax.experimental import pallas as pl
from jax.experimental.pallas import tpu as pltpu
from jax.experimental.pallas import tpu_sc as plsc

B = 16384
ED = 32
HIST = 50
NLIN = 13
HID = 256

NC, NS, L = 2, 16, 16
NW = NC * NS
BPW = B // NW
CB = 32
NCH = BPW // CB


def _sc_gather_pool(user, item, hist_flat, E_user, E_item, E_hist):
    mesh = plsc.VectorSubcoreMesh(core_axis_name="c", subcore_axis_name="s")

    @functools.partial(
        pl.kernel,
        out_type=[
            jax.ShapeDtypeStruct((B, ED), jnp.float32),
            jax.ShapeDtypeStruct((B, ED), jnp.float32),
            jax.ShapeDtypeStruct((B, ED), jnp.float32),
        ],
        mesh=mesh,
        scratch_types=[
            pltpu.VMEM((BPW,), jnp.int32),
            pltpu.VMEM((BPW, ED), jnp.float32),
            pltpu.VMEM((CB * HIST,), jnp.int32),
            pltpu.VMEM((CB * HIST, ED), jnp.float32),
            pltpu.VMEM((CB, ED), jnp.float32),
            pltpu.SemaphoreType.DMA,
        ],
        compiler_params=pltpu.CompilerParams(use_tc_tiling_on_sc=False),
    )
    def k(user_h, item_h, hist_h, eu_h, ei_h, eh_h, out_u, out_i, out_h,
          idx_v, rows_v, hidx_v, hrows_v, acc_v, sem):
        wid = lax.axis_index("s") * NC + lax.axis_index("c")
        base = wid * BPW

        pltpu.sync_copy(user_h.at[pl.ds(base, BPW)], idx_v)
        pltpu.async_copy(eu_h.at[idx_v], rows_v, sem).wait()
        pltpu.sync_copy(rows_v, out_u.at[pl.ds(base, BPW)])

        pltpu.sync_copy(item_h.at[pl.ds(base, BPW)], idx_v)
        pltpu.async_copy(ei_h.at[idx_v], rows_v, sem).wait()
        pltpu.sync_copy(rows_v, out_i.at[pl.ds(base, BPW)])

        def chunk_body(c, carry):
            row0 = base + c * CB
            pltpu.sync_copy(hist_h.at[pl.ds(row0 * HIST, CB * HIST)], hidx_v)
            pltpu.async_copy(eh_h.at[hidx_v], hrows_v, sem).wait()

            def row_body(b, carry2):
                a0 = jnp.zeros((L,), jnp.float32)
                a1 = jnp.zeros((L,), jnp.float32)
                for j in range(HIST):
                    a0 = a0 + hrows_v[b * HIST + j, 0:L]
                    a1 = a1 + hrows_v[b * HIST + j, L:2 * L]
                acc_v[b, 0:L] = a0
                acc_v[b, L:2 * L] = a1
                return carry2

            lax.fori_loop(0, CB, row_body, 0)
            pltpu.sync_copy(acc_v, out_h.at[pl.ds(row0, CB)])
            return carry

        lax.fori_loop(0, NCH, chunk_body, 0)

    return k(user, item, hist_flat, E_user, E_item, E_hist)


BT = 2048


def _tc_mlp(eu, ei, eh, dense, wd_t, bd, w1u, w1i, w1h, w1d, b1, w2_t, b2):
    grid = (B // BT,)

    def body(eu_ref, ei_ref, eh_ref, d_ref, wd_ref, bd_ref,
             w1u_ref, w1i_ref, w1h_ref, w1d_ref, b1_ref, w2_ref, b2_ref,
             o_ref):
        embd = jnp.dot(d_ref[...], wd_ref[...],
                       preferred_element_type=jnp.float32) + bd_ref[...]
        h1 = (jnp.dot(eu_ref[...], w1u_ref[...],
                      preferred_element_type=jnp.float32)
              + jnp.dot(ei_ref[...], w1i_ref[...],
                        preferred_element_type=jnp.float32)
              + jnp.dot(eh_ref[...], w1h_ref[...],
                        preferred_element_type=jnp.float32)
              + jnp.dot(embd, w1d_ref[...],
                        preferred_element_type=jnp.float32)
              + b1_ref[...])
        h1 = jnp.maximum(h1, 0.0)
        o_ref[...] = jnp.dot(h1, w2_ref[...],
                             preferred_element_type=jnp.float32) + b2_ref[...]

    batch_spec = lambda d: pl.BlockSpec((BT, d), lambda i: (i, 0))
    full = lambda a: pl.BlockSpec(a.shape, lambda i: (0,) * a.ndim)

    return pl.pallas_call(
        body,
        grid=grid,
        in_specs=[
            batch_spec(ED), batch_spec(ED), batch_spec(ED), batch_spec(NLIN),
            full(wd_t), full(bd), full(w1u), full(w1i), full(w1h), full(w1d),
            full(b1), full(w2_t), full(b2),
        ],
        out_specs=pl.BlockSpec((BT, 1), lambda i: (i, 0)),
        out_shape=jax.ShapeDtypeStruct((B, 1), jnp.float32),
    )(eu, ei, eh, dense, wd_t, bd, w1u, w1i, w1h, w1d, b1, w2_t, b2)


def kernel(user, item, hist, dense, E_user, E_item, E_hist,
           W_dense, b_dense, W1, b1, W2, b2):
    eu, ei, eh = _sc_gather_pool(
        user.astype(jnp.int32), item.astype(jnp.int32),
        hist.reshape(-1).astype(jnp.int32), E_user, E_item, E_hist)
    w1_t = W1.T
    return _tc_mlp(
        eu, ei, eh, dense,
        W_dense.T, b_dense.reshape(1, ED),
        w1_t[0 * ED:1 * ED], w1_t[1 * ED:2 * ED],
        w1_t[2 * ED:3 * ED], w1_t[3 * ED:4 * ED],
        b1.reshape(1, HID), W2.T, b2.reshape(1, 1))
